# baseline (speedup 1.0000x reference)
import functools

import jax
import jax.numpy as jnp
import numpy as np
from jax.experimental import pallas as pl
from jax.experimental.pallas import tpu as pltpu
from jax.experimental.pallas import tpu_sc as plsc

N_POINTS = 131072
L_LEVELS = 16
FDIM = 2
T_SIZE = 1 << 19
N_MIN = 16
SCALE = 0.5
B_GROWTH = float(np.exp(np.log(2048 * SCALE / N_MIN) / (L_LEVELS - 1)))
RES_LIST = [int(np.floor(N_MIN * (B_GROWTH ** l))) for l in range(L_LEVELS)]
PRIME1 = np.uint32(2654435761)
PRIME2 = np.uint32(805459861)


_NC = 2
_NS = 16
_NW = _NC * _NS
_CH = 64
_P1I = np.int32(np.uint32(2654435761).view(np.int32))
_P2I = np.int32(805459861)


def _sc_encode_body(x0_hbm, x1_hbm, x2_hbm, grid8_hbm, feat_hbm,
                    xb, idxb, lowb, rows, featb, sem):
    cid = jax.lax.axis_index("c")
    sid = jax.lax.axis_index("s")
    wid = sid * _NC + cid
    pw = N_POINTS // _NW
    lanes = jax.lax.iota(jnp.int32, 16)

    def chunk_body(t, carry):
        base = wid * pw + t * _CH
        pltpu.sync_copy(x0_hbm.at[pl.ds(base, _CH)], xb.at[0])
        pltpu.sync_copy(x1_hbm.at[pl.ds(base, _CH)], xb.at[1])
        pltpu.sync_copy(x2_hbm.at[pl.ds(base, _CH)], xb.at[2])

        def s1_body(s, c1):
            sl = pl.ds(s * 16, 16)
            x0 = jnp.clip(xb[0, sl] + SCALE, 0.0, 1.0)
            y0 = jnp.clip(xb[1, sl] + SCALE, 0.0, 1.0)
            z0 = jnp.clip(xb[2, sl] + SCALE, 0.0, 1.0)
            for l in range(L_LEVELS):
                res = float(RES_LIST[l])
                pxi = (x0 * res).astype(jnp.int32)
                pyi = (y0 * res).astype(jnp.int32)
                pzi = (z0 * res).astype(jnp.int32)
                xa = pxi
                xc = pxi + 1
                ya = pyi * _P1I
                yc = ya + _P1I
                za = pzi * _P2I
                zc = za + _P2I
                lofs = l * (T_SIZE // 4)
                for c in range(8):
                    h = (xc if (c & 1) else xa) ^ (yc if (c >> 1) & 1 else ya) \
                        ^ (zc if (c >> 2) & 1 else za)
                    h = h & (T_SIZE - 1)
                    idxb[l * 8 + c, sl] = (h >> 2) + lofs
                    lowb[l * 8 + c, sl] = (h & 3) * 2
            return c1

        jax.lax.fori_loop(0, _CH // 16, s1_body, 0)

        def grp_body(g, c1):
            copies = []
            for i in range(16):
                r = g * 16 + i
                copies.append(pltpu.async_copy(
                    grid8_hbm.at[idxb.at[r]],
                    rows.at[pl.ds(r * _CH, _CH)], sem))
            for cp in copies:
                cp.wait()
            return c1

        jax.lax.fori_loop(0, 128 // 16, grp_body, 0)

        def s2_body(s, c1):
            sl = pl.ds(s * 16, 16)
            x0 = jnp.clip(xb[0, sl] + SCALE, 0.0, 1.0)
            y0 = jnp.clip(xb[1, sl] + SCALE, 0.0, 1.0)
            z0 = jnp.clip(xb[2, sl] + SCALE, 0.0, 1.0)
            prow = s * 16 + lanes
            for l in range(L_LEVELS):
                res = float(RES_LIST[l])
                posx = x0 * res
                posy = y0 * res
                posz = z0 * res
                wx1 = posx - posx.astype(jnp.int32).astype(jnp.float32)
                wy1 = posy - posy.astype(jnp.int32).astype(jnp.float32)
                wz1 = posz - posz.astype(jnp.int32).astype(jnp.float32)
                wx0 = 1.0 - wx1
                wy0 = 1.0 - wy1
                wz0 = 1.0 - wz1
                acc0 = jnp.zeros((16,), jnp.float32)
                acc1 = jnp.zeros((16,), jnp.float32)
                for c in range(8):
                    wt = ((wx1 if (c & 1) else wx0)
                          * (wy1 if (c >> 1) & 1 else wy0)
                          * (wz1 if (c >> 2) & 1 else wz0))
                    r = l * 8 + c
                    ridx = r * _CH + prow
                    lcol = lowb[r, sl]
                    f0 = plsc.load_gather(rows, [ridx, lcol])
                    f1 = plsc.load_gather(rows, [ridx, lcol + 1])
                    acc0 = acc0 + wt * f0
                    acc1 = acc1 + wt * f1
                plsc.store_scatter(featb, [prow, jnp.full((16,), 2 * l, jnp.int32)], acc0)
                plsc.store_scatter(featb, [prow, jnp.full((16,), 2 * l + 1, jnp.int32)], acc1)
            return c1

        jax.lax.fori_loop(0, _CH // 16, s2_body, 0)
        pltpu.sync_copy(featb, feat_hbm.at[pl.ds(base, _CH)])
        return carry

    jax.lax.fori_loop(0, pw // _CH, chunk_body, 0)


_sc_encode = pl.kernel(
    _sc_encode_body,
    out_type=jax.ShapeDtypeStruct((N_POINTS, 2 * L_LEVELS), jnp.float32),
    mesh=plsc.VectorSubcoreMesh(core_axis_name="c", subcore_axis_name="s"),
    compiler_params=pltpu.CompilerParams(needs_layout_passes=False,
                                         use_tc_tiling_on_sc=False),
    scratch_types=[
        pltpu.VMEM((3, _CH), jnp.float32),
        pltpu.VMEM((128, _CH), jnp.int32),
        pltpu.VMEM((128, _CH), jnp.int32),
        pltpu.VMEM((128 * _CH, 8), jnp.float32),
        pltpu.VMEM((_CH, 2 * L_LEVELS), jnp.float32),
        pltpu.SemaphoreType.DMA,
    ],
)


_BLK = 4096


def _head_kernel(feat_ref, d_ref,
                 gw0, gw1, gw2, a0, a1, a2, b0, b1, b2, geo0, geo1, r0, r1, r2,
                 sig_ref, rgb_ref, gates_ref, load_ref, tidx_ref):
    i = pl.program_id(0)
    feat = feat_ref[...]

    def dot(x, w):
        return jax.lax.dot_general(x, w[...], (((1,), (0,)), ((), ())),
                                   preferred_element_type=jnp.float32)

    relu = lambda v: jnp.maximum(v, 0.0)

    g = relu(dot(relu(dot(feat, gw0)), gw1))
    logits = dot(g, gw2)
    l0 = logits[:, 0]
    l1 = logits[:, 1]
    sel = l1 > l0
    sel_f = sel.astype(jnp.float32)

    tidx_ref[...] = sel.astype(jnp.int32)[:, None]
    gates_ref[...] = jnp.stack([1.0 - sel_f, sel_f], axis=1)

    cnt1 = jnp.sum(sel_f)
    cnt = jnp.stack([jnp.float32(feat.shape[0]) - cnt1, cnt1])

    @pl.when(i == 0)
    def _():
        load_ref[...] = jnp.zeros_like(load_ref)

    load_ref[...] += cnt

    e0 = dot(relu(dot(relu(dot(feat, a0)), a1)), a2)
    e1 = dot(relu(dot(relu(dot(feat, b0)), b1)), b2)
    post = jnp.where(sel[:, None], e1, e0)

    h = dot(relu(dot(post, geo0)), geo1)
    sig_ref[...] = jnp.exp(h[:, 0])

    d = d_ref[...]
    dx = d[:, 0]
    dy = d[:, 1]
    dz = d[:, 2]
    inv = jax.lax.rsqrt(dx * dx + dy * dy + dz * dz)
    nrm = 1.0 / (1.0 / inv + 1e-8)
    x = dx * nrm
    y = dy * nrm
    z = dz * nrm
    xx = x * x
    yy = y * y
    zz = z * z
    xy = x * y
    yz = y * z
    xz = x * z
    sh_cols = [
        jnp.full_like(x, 0.28209479177387814),
        -0.48860251190291987 * y,
        0.48860251190291987 * z,
        -0.48860251190291987 * x,
        1.0925484305920792 * xy,
        -1.0925484305920792 * yz,
        0.94617469575755997 * zz - 0.31539156525251999,
        -1.0925484305920792 * xz,
        0.54627421529603959 * (xx - yy),
        0.59004358992664352 * y * (-3.0 * xx + yy),
        2.8906114426405538 * xy * z,
        0.45704579946446572 * y * (1.0 - 5.0 * zz),
        0.3731763325901154 * z * (5.0 * zz - 3.0),
        0.45704579946446572 * x * (1.0 - 5.0 * zz),
        1.4453057213202769 * z * (xx - yy),
        0.59004358992664352 * x * (-xx + 3.0 * yy),
    ]
    sh = jnp.stack(sh_cols, axis=1)
    rgb_in = jnp.concatenate([sh, h[:, 1:]], axis=1)
    r = dot(relu(dot(relu(dot(rgb_in, r0)), r1)), r2)
    rgb_ref[...] = jax.nn.sigmoid(r)


def _head(feat, d, weights):
    n = feat.shape[0]
    grid_n = n // _BLK
    row_spec = lambda width: pl.BlockSpec((_BLK, width), lambda i: (i, 0))
    full = lambda a: pl.BlockSpec(a.shape, lambda i: (0,) * a.ndim)
    out_shapes = (
        jax.ShapeDtypeStruct((n,), jnp.float32),
        jax.ShapeDtypeStruct((n, 3), jnp.float32),
        jax.ShapeDtypeStruct((n, 2), jnp.float32),
        jax.ShapeDtypeStruct((2,), jnp.float32),
        jax.ShapeDtypeStruct((n, 1), jnp.int32),
    )
    out_specs = (
        pl.BlockSpec((_BLK,), lambda i: (i,)),
        row_spec(3),
        row_spec(2),
        pl.BlockSpec((2,), lambda i: (0,)),
        row_spec(1),
    )
    return pl.pallas_call(
        _head_kernel,
        grid=(grid_n,),
        in_specs=[row_spec(32), row_spec(3)] + [full(w) for w in weights],
        out_specs=out_specs,
        out_shape=out_shapes,
    )(feat, d, *weights)


def kernel(x, d, grid, gate_w0, gate_w1, gate_w2, i0w0, i0w1, i0w2,
           i1w0, i1w1, i1w2, geo_w0, geo_w1, rgb_w0, rgb_w1, rgb_w2):
    grid8 = grid.reshape(L_LEVELS * T_SIZE // 4, 4 * FDIM)
    feat = _sc_encode(x[:, 0], x[:, 1], x[:, 2], grid8)
    weights = (gate_w0, gate_w1, gate_w2, i0w0, i0w1, i0w2, i1w0, i1w1, i1w2,
               geo_w0, geo_w1, rgb_w0, rgb_w1, rgb_w2)
    sigmas, rgbs, gates, load, top_idx = _head(feat, d, weights)
    return (sigmas, rgbs, gates, load, top_idx)


# speedup vs baseline: 4.9988x; 4.9988x over previous
import functools

import jax
import jax.numpy as jnp
import numpy as np
from jax.experimental import pallas as pl
from jax.experimental.pallas import tpu as pltpu
from jax.experimental.pallas import tpu_sc as plsc

N_POINTS = 131072
L_LEVELS = 16
FDIM = 2
T_SIZE = 1 << 19
N_MIN = 16
SCALE = 0.5
B_GROWTH = float(np.exp(np.log(2048 * SCALE / N_MIN) / (L_LEVELS - 1)))
RES_LIST = [int(np.floor(N_MIN * (B_GROWTH ** l))) for l in range(L_LEVELS)]
PRIME1 = np.uint32(2654435761)
PRIME2 = np.uint32(805459861)


_NC = 2
_NS = 16
_NW = _NC * _NS
_CH = 32
_P1I = np.int32(np.uint32(2654435761).view(np.int32))
_P2I = np.int32(805459861)


def _sc_encode_body(x0_hbm, x1_hbm, x2_hbm, grid8_hbm, feat_hbm,
                    xb, idxb, lowb, rows, featb, sem):
    cid = jax.lax.axis_index("c")
    sid = jax.lax.axis_index("s")
    wid = sid * _NC + cid
    pw = N_POINTS // _NW
    lanes = jax.lax.iota(jnp.int32, 16)

    def chunk_body(t, carry):
        base = wid * pw + t * _CH
        pltpu.sync_copy(x0_hbm.at[pl.ds(base, _CH)], xb.at[0])
        pltpu.sync_copy(x1_hbm.at[pl.ds(base, _CH)], xb.at[1])
        pltpu.sync_copy(x2_hbm.at[pl.ds(base, _CH)], xb.at[2])

        def s1_body(s, c1):
            sl = pl.ds(s * 16, 16)
            x0 = jnp.clip(xb[0, sl] + SCALE, 0.0, 1.0)
            y0 = jnp.clip(xb[1, sl] + SCALE, 0.0, 1.0)
            z0 = jnp.clip(xb[2, sl] + SCALE, 0.0, 1.0)
            for l in range(L_LEVELS):
                res = float(RES_LIST[l])
                pxi = (x0 * res).astype(jnp.int32)
                pyi = (y0 * res).astype(jnp.int32)
                pzi = (z0 * res).astype(jnp.int32)
                xa = pxi
                xc = pxi + 1
                ya = pyi * _P1I
                yc = ya + _P1I
                za = pzi * _P2I
                zc = za + _P2I
                for c in range(8):
                    h = (xc if (c & 1) else xa) ^ (yc if (c >> 1) & 1 else ya) \
                        ^ (zc if (c >> 2) & 1 else za)
                    h = h & (T_SIZE - 1)
                    j = l * 8 + c
                    rf0 = (h >> 7) * 32 + ((h >> 3) & 15) + l * 131072
                    idxb[pl.ds(j * _CH + s * 16, 16)] = rf0
                    idxb[pl.ds((128 + j) * _CH + s * 16, 16)] = rf0 + 16
                    lowb[j, sl] = h & 7
            return c1

        jax.lax.fori_loop(0, _CH // 16, s1_body, 0)

        pltpu.async_copy(grid8_hbm.at[idxb], rows, sem).wait()

        def s2_body(s, c1):
            sl = pl.ds(s * 16, 16)
            x0 = jnp.clip(xb[0, sl] + SCALE, 0.0, 1.0)
            y0 = jnp.clip(xb[1, sl] + SCALE, 0.0, 1.0)
            z0 = jnp.clip(xb[2, sl] + SCALE, 0.0, 1.0)
            prow = s * 16 + lanes
            for l in range(L_LEVELS):
                res = float(RES_LIST[l])
                posx = x0 * res
                posy = y0 * res
                posz = z0 * res
                wx1 = posx - posx.astype(jnp.int32).astype(jnp.float32)
                wy1 = posy - posy.astype(jnp.int32).astype(jnp.float32)
                wz1 = posz - posz.astype(jnp.int32).astype(jnp.float32)
                wx0 = 1.0 - wx1
                wy0 = 1.0 - wy1
                wz0 = 1.0 - wz1
                acc0 = jnp.zeros((16,), jnp.float32)
                acc1 = jnp.zeros((16,), jnp.float32)
                for c in range(8):
                    wt = ((wx1 if (c & 1) else wx0)
                          * (wy1 if (c >> 1) & 1 else wy0)
                          * (wz1 if (c >> 2) & 1 else wz0))
                    j = l * 8 + c
                    lcol = lowb[j, sl]
                    f0 = plsc.load_gather(rows, [j * _CH + prow, lcol])
                    f1 = plsc.load_gather(rows, [(128 + j) * _CH + prow, lcol])
                    acc0 = acc0 + wt * f0
                    acc1 = acc1 + wt * f1
                plsc.store_scatter(featb, [prow, jnp.full((16,), 2 * l, jnp.int32)], acc0)
                plsc.store_scatter(featb, [prow, jnp.full((16,), 2 * l + 1, jnp.int32)], acc1)
            return c1

        jax.lax.fori_loop(0, _CH // 16, s2_body, 0)
        pltpu.sync_copy(featb, feat_hbm.at[pl.ds(base, _CH)])
        return carry

    jax.lax.fori_loop(0, pw // _CH, chunk_body, 0)


_sc_encode = pl.kernel(
    _sc_encode_body,
    out_type=jax.ShapeDtypeStruct((N_POINTS, 2 * L_LEVELS), jnp.float32),
    mesh=plsc.VectorSubcoreMesh(core_axis_name="c", subcore_axis_name="s"),
    compiler_params=pltpu.CompilerParams(needs_layout_passes=False,
                                         use_tc_tiling_on_sc=False),
    scratch_types=[
        pltpu.VMEM((3, _CH), jnp.float32),
        pltpu.VMEM((256 * _CH,), jnp.int32),
        pltpu.VMEM((128, _CH), jnp.int32),
        pltpu.VMEM((256 * _CH, 8), jnp.float32),
        pltpu.VMEM((_CH, 2 * L_LEVELS), jnp.float32),
        pltpu.SemaphoreType.DMA,
    ],
)


_BLK = 4096


def _head_kernel(feat_ref, d_ref,
                 gw0, gw1, gw2, a0, a1, a2, b0, b1, b2, geo0, geo1, r0, r1, r2,
                 sig_ref, rgb_ref, gates_ref, load_ref, tidx_ref):
    i = pl.program_id(0)
    feat = feat_ref[...]

    def dot(x, w):
        return jax.lax.dot_general(x, w[...], (((1,), (0,)), ((), ())),
                                   preferred_element_type=jnp.float32)

    relu = lambda v: jnp.maximum(v, 0.0)

    g = relu(dot(relu(dot(feat, gw0)), gw1))
    logits = dot(g, gw2)
    l0 = logits[:, 0]
    l1 = logits[:, 1]
    sel = l1 > l0
    sel_f = sel.astype(jnp.float32)

    tidx_ref[...] = sel.astype(jnp.int32)[:, None]
    gates_ref[...] = jnp.stack([1.0 - sel_f, sel_f], axis=1)

    cnt1 = jnp.sum(sel_f)
    cnt = jnp.stack([jnp.float32(feat.shape[0]) - cnt1, cnt1])

    @pl.when(i == 0)
    def _():
        load_ref[...] = jnp.zeros_like(load_ref)

    load_ref[...] += cnt

    e0 = dot(relu(dot(relu(dot(feat, a0)), a1)), a2)
    e1 = dot(relu(dot(relu(dot(feat, b0)), b1)), b2)
    post = jnp.where(sel[:, None], e1, e0)

    h = dot(relu(dot(post, geo0)), geo1)
    sig_ref[...] = jnp.exp(h[:, 0])

    d = d_ref[...]
    dx = d[:, 0]
    dy = d[:, 1]
    dz = d[:, 2]
    inv = jax.lax.rsqrt(dx * dx + dy * dy + dz * dz)
    nrm = 1.0 / (1.0 / inv + 1e-8)
    x = dx * nrm
    y = dy * nrm
    z = dz * nrm
    xx = x * x
    yy = y * y
    zz = z * z
    xy = x * y
    yz = y * z
    xz = x * z
    sh_cols = [
        jnp.full_like(x, 0.28209479177387814),
        -0.48860251190291987 * y,
        0.48860251190291987 * z,
        -0.48860251190291987 * x,
        1.0925484305920792 * xy,
        -1.0925484305920792 * yz,
        0.94617469575755997 * zz - 0.31539156525251999,
        -1.0925484305920792 * xz,
        0.54627421529603959 * (xx - yy),
        0.59004358992664352 * y * (-3.0 * xx + yy),
        2.8906114426405538 * xy * z,
        0.45704579946446572 * y * (1.0 - 5.0 * zz),
        0.3731763325901154 * z * (5.0 * zz - 3.0),
        0.45704579946446572 * x * (1.0 - 5.0 * zz),
        1.4453057213202769 * z * (xx - yy),
        0.59004358992664352 * x * (-xx + 3.0 * yy),
    ]
    sh = jnp.stack(sh_cols, axis=1)
    rgb_in = jnp.concatenate([sh, h[:, 1:]], axis=1)
    r = dot(relu(dot(relu(dot(rgb_in, r0)), r1)), r2)
    rgb_ref[...] = jax.nn.sigmoid(r)


def _head(feat, d, weights):
    n = feat.shape[0]
    grid_n = n // _BLK
    row_spec = lambda width: pl.BlockSpec((_BLK, width), lambda i: (i, 0))
    full = lambda a: pl.BlockSpec(a.shape, lambda i: (0,) * a.ndim)
    out_shapes = (
        jax.ShapeDtypeStruct((n,), jnp.float32),
        jax.ShapeDtypeStruct((n, 3), jnp.float32),
        jax.ShapeDtypeStruct((n, 2), jnp.float32),
        jax.ShapeDtypeStruct((2,), jnp.float32),
        jax.ShapeDtypeStruct((n, 1), jnp.int32),
    )
    out_specs = (
        pl.BlockSpec((_BLK,), lambda i: (i,)),
        row_spec(3),
        row_spec(2),
        pl.BlockSpec((2,), lambda i: (0,)),
        row_spec(1),
    )
    return pl.pallas_call(
        _head_kernel,
        grid=(grid_n,),
        in_specs=[row_spec(32), row_spec(3)] + [full(w) for w in weights],
        out_specs=out_specs,
        out_shape=out_shapes,
    )(feat, d, *weights)


def kernel(x, d, grid, gate_w0, gate_w1, gate_w2, i0w0, i0w1, i0w2,
           i1w0, i1w1, i1w2, geo_w0, geo_w1, rgb_w0, rgb_w1, rgb_w2):
    grid8 = (grid.reshape(L_LEVELS, T_SIZE // 128, 128, FDIM)
             .transpose(0, 1, 3, 2)
             .reshape(L_LEVELS * T_SIZE * FDIM // 8, 8))
    feat = _sc_encode(x[:, 0], x[:, 1], x[:, 2], grid8)
    weights = (gate_w0, gate_w1, gate_w2, i0w0, i0w1, i0w2, i1w0, i1w1, i1w2,
               geo_w0, geo_w1, rgb_w0, rgb_w1, rgb_w2)
    sigmas, rgbs, gates, load, top_idx = _head(feat, d, weights)
    return (sigmas, rgbs, gates, load, top_idx)


# speedup vs baseline: 5.9744x; 1.1952x over previous
import functools

import jax
import jax.numpy as jnp
import numpy as np
from jax.experimental import pallas as pl
from jax.experimental.pallas import tpu as pltpu
from jax.experimental.pallas import tpu_sc as plsc

N_POINTS = 131072
L_LEVELS = 16
FDIM = 2
T_SIZE = 1 << 19
N_MIN = 16
SCALE = 0.5
B_GROWTH = float(np.exp(np.log(2048 * SCALE / N_MIN) / (L_LEVELS - 1)))
RES_LIST = [int(np.floor(N_MIN * (B_GROWTH ** l))) for l in range(L_LEVELS)]
PRIME1 = np.uint32(2654435761)
PRIME2 = np.uint32(805459861)


_NC = 2
_NS = 16
_NW = _NC * _NS
_CH = 16
_P1I = np.int32(np.uint32(2654435761).view(np.int32))
_P2I = np.int32(805459861)


def _sc_encode_body(x0_hbm, x1_hbm, x2_hbm, grid8_hbm, feat_hbm,
                    xb0, xb1, idxb0, idxb1, lowb0, lowb1, rows0, rows1,
                    featb, sem0, sem1):
    cid = jax.lax.axis_index("c")
    sid = jax.lax.axis_index("s")
    wid = sid * _NC + cid
    pw = N_POINTS // _NW
    lanes = jax.lax.iota(jnp.int32, 16)

    def load_x(t, xb):
        base = wid * pw + t * _CH
        pltpu.sync_copy(x0_hbm.at[pl.ds(base, _CH)], xb.at[0])
        pltpu.sync_copy(x1_hbm.at[pl.ds(base, _CH)], xb.at[1])
        pltpu.sync_copy(x2_hbm.at[pl.ds(base, _CH)], xb.at[2])

    def clipped(xb):
        x0 = jnp.clip(xb[0] + SCALE, 0.0, 1.0)
        y0 = jnp.clip(xb[1] + SCALE, 0.0, 1.0)
        z0 = jnp.clip(xb[2] + SCALE, 0.0, 1.0)
        return x0, y0, z0

    def pass1(xb, idxb, lowb):
        x0, y0, z0 = clipped(xb)
        for l in range(L_LEVELS):
            res = float(RES_LIST[l])
            pxi = (x0 * res).astype(jnp.int32)
            pyi = (y0 * res).astype(jnp.int32)
            pzi = (z0 * res).astype(jnp.int32)
            xa = pxi
            xc = pxi + 1
            ya = pyi * _P1I
            yc = ya + _P1I
            za = pzi * _P2I
            zc = za + _P2I
            for c in range(8):
                h = (xc if (c & 1) else xa) ^ (yc if (c >> 1) & 1 else ya) \
                    ^ (zc if (c >> 2) & 1 else za)
                h = h & (T_SIZE - 1)
                j = l * 8 + c
                rf0 = (h >> 7) * 32 + ((h >> 3) & 15) + l * 131072
                idxb[pl.ds(j * _CH, _CH)] = rf0
                idxb[pl.ds((128 + j) * _CH, _CH)] = rf0 + 16
                lowb[j] = h & 7

    def fire(idxb, rows, sem):
        pltpu.async_copy(grid8_hbm.at[idxb], rows, sem)

    def wait(idxb, rows, sem):
        pltpu.make_async_copy(grid8_hbm.at[idxb], rows, sem).wait()

    def pass2(t, xb, lowb, rows):
        base = wid * pw + t * _CH
        x0, y0, z0 = clipped(xb)
        for l in range(L_LEVELS):
            res = float(RES_LIST[l])
            posx = x0 * res
            posy = y0 * res
            posz = z0 * res
            wx1 = posx - posx.astype(jnp.int32).astype(jnp.float32)
            wy1 = posy - posy.astype(jnp.int32).astype(jnp.float32)
            wz1 = posz - posz.astype(jnp.int32).astype(jnp.float32)
            wx0 = 1.0 - wx1
            wy0 = 1.0 - wy1
            wz0 = 1.0 - wz1
            acc0 = jnp.zeros((16,), jnp.float32)
            acc1 = jnp.zeros((16,), jnp.float32)
            for c in range(8):
                wt = ((wx1 if (c & 1) else wx0)
                      * (wy1 if (c >> 1) & 1 else wy0)
                      * (wz1 if (c >> 2) & 1 else wz0))
                j = l * 8 + c
                lcol = lowb[j]
                f0 = plsc.load_gather(rows, [j * _CH + lanes, lcol])
                f1 = plsc.load_gather(rows, [(128 + j) * _CH + lanes, lcol])
                acc0 = acc0 + wt * f0
                acc1 = acc1 + wt * f1
            plsc.store_scatter(featb, [lanes, jnp.full((16,), 2 * l, jnp.int32)], acc0)
            plsc.store_scatter(featb, [lanes, jnp.full((16,), 2 * l + 1, jnp.int32)], acc1)
        pltpu.sync_copy(featb, feat_hbm.at[pl.ds(base, _CH)])

    ng = pw // _CH // 2

    load_x(0, xb0)
    pass1(xb0, idxb0, lowb0)
    fire(idxb0, rows0, sem0)

    def g_body(g, carry):
        t0 = 2 * g
        load_x(t0 + 1, xb1)
        pass1(xb1, idxb1, lowb1)
        fire(idxb1, rows1, sem1)
        wait(idxb0, rows0, sem0)
        pass2(t0, xb0, lowb0, rows0)

        @pl.when(g < ng - 1)
        def _():
            load_x(t0 + 2, xb0)
            pass1(xb0, idxb0, lowb0)
            fire(idxb0, rows0, sem0)

        wait(idxb1, rows1, sem1)
        pass2(t0 + 1, xb1, lowb1, rows1)
        return carry

    jax.lax.fori_loop(0, ng, g_body, 0)


_sc_encode = pl.kernel(
    _sc_encode_body,
    out_type=jax.ShapeDtypeStruct((N_POINTS, 2 * L_LEVELS), jnp.float32),
    mesh=plsc.VectorSubcoreMesh(core_axis_name="c", subcore_axis_name="s"),
    compiler_params=pltpu.CompilerParams(needs_layout_passes=False,
                                         use_tc_tiling_on_sc=False),
    scratch_types=[
        pltpu.VMEM((3, _CH), jnp.float32),
        pltpu.VMEM((3, _CH), jnp.float32),
        pltpu.VMEM((256 * _CH,), jnp.int32),
        pltpu.VMEM((256 * _CH,), jnp.int32),
        pltpu.VMEM((128, _CH), jnp.int32),
        pltpu.VMEM((128, _CH), jnp.int32),
        pltpu.VMEM((256 * _CH, 8), jnp.float32),
        pltpu.VMEM((256 * _CH, 8), jnp.float32),
        pltpu.VMEM((_CH, 2 * L_LEVELS), jnp.float32),
        pltpu.SemaphoreType.DMA,
        pltpu.SemaphoreType.DMA,
    ],
)


_BLK = 4096


def _head_kernel(feat_ref, d_ref,
                 gw0, gw1, gw2, a0, a1, a2, b0, b1, b2, geo0, geo1, r0, r1, r2,
                 sig_ref, rgb_ref, gates_ref, load_ref, tidx_ref):
    i = pl.program_id(0)
    feat = feat_ref[...]

    def dot(x, w):
        return jax.lax.dot_general(x, w[...], (((1,), (0,)), ((), ())),
                                   preferred_element_type=jnp.float32)

    relu = lambda v: jnp.maximum(v, 0.0)

    g = relu(dot(relu(dot(feat, gw0)), gw1))
    logits = dot(g, gw2)
    l0 = logits[:, 0]
    l1 = logits[:, 1]
    sel = l1 > l0
    sel_f = sel.astype(jnp.float32)

    tidx_ref[...] = sel.astype(jnp.int32)[:, None]
    gates_ref[...] = jnp.stack([1.0 - sel_f, sel_f], axis=1)

    cnt1 = jnp.sum(sel_f)
    cnt = jnp.stack([jnp.float32(feat.shape[0]) - cnt1, cnt1])

    @pl.when(i == 0)
    def _():
        load_ref[...] = jnp.zeros_like(load_ref)

    load_ref[...] += cnt

    e0 = dot(relu(dot(relu(dot(feat, a0)), a1)), a2)
    e1 = dot(relu(dot(relu(dot(feat, b0)), b1)), b2)
    post = jnp.where(sel[:, None], e1, e0)

    h = dot(relu(dot(post, geo0)), geo1)
    sig_ref[...] = jnp.exp(h[:, 0])

    d = d_ref[...]
    dx = d[:, 0]
    dy = d[:, 1]
    dz = d[:, 2]
    inv = jax.lax.rsqrt(dx * dx + dy * dy + dz * dz)
    nrm = 1.0 / (1.0 / inv + 1e-8)
    x = dx * nrm
    y = dy * nrm
    z = dz * nrm
    xx = x * x
    yy = y * y
    zz = z * z
    xy = x * y
    yz = y * z
    xz = x * z
    sh_cols = [
        jnp.full_like(x, 0.28209479177387814),
        -0.48860251190291987 * y,
        0.48860251190291987 * z,
        -0.48860251190291987 * x,
        1.0925484305920792 * xy,
        -1.0925484305920792 * yz,
        0.94617469575755997 * zz - 0.31539156525251999,
        -1.0925484305920792 * xz,
        0.54627421529603959 * (xx - yy),
        0.59004358992664352 * y * (-3.0 * xx + yy),
        2.8906114426405538 * xy * z,
        0.45704579946446572 * y * (1.0 - 5.0 * zz),
        0.3731763325901154 * z * (5.0 * zz - 3.0),
        0.45704579946446572 * x * (1.0 - 5.0 * zz),
        1.4453057213202769 * z * (xx - yy),
        0.59004358992664352 * x * (-xx + 3.0 * yy),
    ]
    sh = jnp.stack(sh_cols, axis=1)
    rgb_in = jnp.concatenate([sh, h[:, 1:]], axis=1)
    r = dot(relu(dot(relu(dot(rgb_in, r0)), r1)), r2)
    rgb_ref[...] = jax.nn.sigmoid(r)


def _head(feat, d, weights):
    n = feat.shape[0]
    grid_n = n // _BLK
    row_spec = lambda width: pl.BlockSpec((_BLK, width), lambda i: (i, 0))
    full = lambda a: pl.BlockSpec(a.shape, lambda i: (0,) * a.ndim)
    out_shapes = (
        jax.ShapeDtypeStruct((n,), jnp.float32),
        jax.ShapeDtypeStruct((n, 3), jnp.float32),
        jax.ShapeDtypeStruct((n, 2), jnp.float32),
        jax.ShapeDtypeStruct((2,), jnp.float32),
        jax.ShapeDtypeStruct((n, 1), jnp.int32),
    )
    out_specs = (
        pl.BlockSpec((_BLK,), lambda i: (i,)),
        row_spec(3),
        row_spec(2),
        pl.BlockSpec((2,), lambda i: (0,)),
        row_spec(1),
    )
    return pl.pallas_call(
        _head_kernel,
        grid=(grid_n,),
        in_specs=[row_spec(32), row_spec(3)] + [full(w) for w in weights],
        out_specs=out_specs,
        out_shape=out_shapes,
    )(feat, d, *weights)


def kernel(x, d, grid, gate_w0, gate_w1, gate_w2, i0w0, i0w1, i0w2,
           i1w0, i1w1, i1w2, geo_w0, geo_w1, rgb_w0, rgb_w1, rgb_w2):
    grid8 = (grid.reshape(L_LEVELS, T_SIZE // 128, 128, FDIM)
             .transpose(0, 1, 3, 2)
             .reshape(L_LEVELS * T_SIZE * FDIM // 8, 8))
    feat = _sc_encode(x[:, 0], x[:, 1], x[:, 2], grid8)
    weights = (gate_w0, gate_w1, gate_w2, i0w0, i0w1, i0w2, i1w0, i1w1, i1w2,
               geo_w0, geo_w1, rgb_w0, rgb_w1, rgb_w2)
    sigmas, rgbs, gates, load, top_idx = _head(feat, d, weights)
    return (sigmas, rgbs, gates, load, top_idx)
